# baseline (speedup 1.0000x reference)
import functools

import jax
import jax.numpy as jnp
from jax import lax
from jax.experimental import pallas as pl
from jax.experimental.pallas import tpu as pltpu
from jax.experimental.pallas import tpu_sc as plsc

B = 8
NC = 2
NS = 16
NW = NC * NS
L = 16
N_SC = 204800


def _sc_pass1(features, seg, start, n_sc):
    N, C = features.shape
    G = C // L
    RW = n_sc // NW
    CH = 400
    NCH = RW // CH
    UR = 8
    assert RW % CH == 0 and NCH % 2 == 0 and CH % UR == 0

    mesh = plsc.VectorSubcoreMesh(core_axis_name="c", subcore_axis_name="s",
                                  num_cores=NC, num_subcores=NS)

    @functools.partial(
        pl.kernel,
        out_type=jax.ShapeDtypeStruct((NC, 3 * B, C), jnp.float32),
        mesh=mesh,
        scratch_types=[
            pltpu.VMEM((RW + L,), jnp.int32),
            pltpu.VMEM((CH, C), jnp.float32),
            pltpu.VMEM((CH, C), jnp.float32),
            pltpu.VMEM((3 * B, C), jnp.float32),
            pltpu.VMEM_SHARED((NS, 3 * B, C), jnp.float32),
            pltpu.VMEM((3 * B, C), jnp.float32),
            pltpu.SemaphoreType.DMA,
            pltpu.SemaphoreType.DMA,
        ],
    )
    def pass1(feat_hbm, seg_hbm, stats_o,
              ids_v, buf0, buf1, acc, shr, tmp, sem0, sem1):
        ci = lax.axis_index("c")
        si = lax.axis_index("s")
        wid = si * NC + ci
        base = start + wid * RW

        pltpu.sync_copy(seg_hbm.at[pl.ds(base, RW)], ids_v.at[pl.ds(0, RW)])

        zero = jnp.zeros((L,), jnp.float32)
        ninf = jnp.full((L,), -jnp.inf, jnp.float32)
        for b in range(2 * B):
            for j in range(G):
                acc[b, pl.ds(L * j, L)] = zero
        for b in range(2 * B, 3 * B):
            for j in range(G):
                acc[b, pl.ds(L * j, L)] = ninf

        def process(buf, c):
            s0 = ids_v[pl.ds(c * CH, L)][0]
            s1 = ids_v[pl.ds(c * CH + (CH - L), L)][L - 1]

            @pl.when(s0 == s1)
            def _fast():
                def row_body(rr, carry):
                    fs, fm = carry
                    r0 = rr * UR
                    for k in range(UR):
                        vals = tuple(buf[r0 + k, pl.ds(L * j, L)]
                                     for j in range(G))
                        fs = tuple(fs[j] + vals[j] for j in range(G))
                        fm = tuple(jnp.maximum(fm[j], vals[j])
                                   for j in range(G))
                    return fs, fm

                init = (tuple(zero for _ in range(G)),
                        tuple(ninf for _ in range(G)))
                fs, fm = lax.fori_loop(0, CH // UR, row_body, init)
                for j in range(G):
                    sl = pl.ds(L * j, L)
                    acc[s0, sl] += fs[j]
                    acc[2 * B + s0, sl] = jnp.maximum(acc[2 * B + s0, sl],
                                                      fm[j])
                acc[B + s0, pl.ds(0, L)] += jnp.float32(CH)

            @pl.when(s0 != s1)
            def _boundary():
                def row_body(r, carry):
                    sid = ids_v[pl.ds(c * CH + r, L)][0]
                    for j in range(G):
                        sl = pl.ds(L * j, L)
                        v = buf[r, sl]
                        acc[sid, sl] += v
                        acc[2 * B + sid, sl] = jnp.maximum(
                            acc[2 * B + sid, sl], v)
                    acc[B + sid, pl.ds(0, L)] += 1.0
                    return carry

                lax.fori_loop(0, CH, row_body, 0)

        pltpu.async_copy(feat_hbm.at[pl.ds(base, CH), :], buf0, sem0)

        def chunk_pair(i, carry):
            c0 = i * 2
            c1 = c0 + 1

            pltpu.async_copy(
                feat_hbm.at[pl.ds(base + c1 * CH, CH), :], buf1, sem1)
            pltpu.make_async_copy(
                feat_hbm.at[pl.ds(base, CH), :], buf0, sem0).wait()
            process(buf0, c0)

            @pl.when(c1 + 1 < NCH)
            def _next():
                pltpu.async_copy(
                    feat_hbm.at[pl.ds(base + (c1 + 1) * CH, CH), :],
                    buf0, sem0)

            pltpu.make_async_copy(
                feat_hbm.at[pl.ds(base, CH), :], buf1, sem1).wait()
            process(buf1, c1)
            return carry

        lax.fori_loop(0, NCH // 2, chunk_pair, 0)

        pltpu.sync_copy(acc, shr.at[si])
        plsc.subcore_barrier()

        @pl.when(si == 0)
        def _reduce_and_emit():
            def tile_body(t, carry):
                pltpu.sync_copy(shr.at[t], tmp)
                for b in range(2 * B):
                    for j in range(G):
                        sl = pl.ds(L * j, L)
                        acc[b, sl] += tmp[b, sl]
                for b in range(2 * B, 3 * B):
                    for j in range(G):
                        sl = pl.ds(L * j, L)
                        acc[b, sl] = jnp.maximum(acc[b, sl], tmp[b, sl])
                return carry

            lax.fori_loop(1, NS, tile_body, 0)
            pltpu.sync_copy(acc, stats_o.at[ci])

    return pass1(features, seg)


def _tc_pass1_body(seg_ref, feat_ref, stats_ref):
    i = pl.program_id(0)
    R = feat_ref.shape[0]
    C = feat_ref.shape[1]

    @pl.when(i == 0)
    def _init():
        stats_ref[pl.ds(0, 2 * B), :] = jnp.zeros((2 * B, C), jnp.float32)
        stats_ref[pl.ds(2 * B, B), :] = jnp.full((B, C), -jnp.inf,
                                                 jnp.float32)

    feat = feat_ref[...]
    segv = seg_ref[0, 0, :]

    oh = (lax.broadcasted_iota(jnp.int32, (B, R), 0)
          == segv[None, :]).astype(jnp.float32)
    stats_ref[pl.ds(0, B), :] += lax.dot(oh, feat,
                                         preferred_element_type=jnp.float32)
    stats_ref[pl.ds(B, B), :] += jnp.broadcast_to(
        jnp.sum(oh, axis=1, keepdims=True), (B, C))

    T = R // 8
    m1 = jnp.max(feat.reshape(T, 8, C), axis=1)
    rowt = lax.broadcasted_iota(jnp.int32, (T, C), 0)
    row8 = lax.broadcasted_iota(jnp.int32, (8, C), 0)
    los = [jnp.sum((segv < b).astype(jnp.int32)) for b in range(B)]
    his = los[1:] + [jnp.sum((segv <= B - 1).astype(jnp.int32))]
    mxs = []
    for b in range(B):
        lo, hi = los[b], his[b]
        tlo = (lo + 7) // 8
        thi = hi // 8
        main = jnp.max(jnp.where((rowt >= tlo) & (rowt < thi), m1, -jnp.inf),
                       axis=0, keepdims=True)
        elo = lo // 8
        ehi = jnp.maximum(hi - 1, 0) // 8
        e0 = feat_ref[pl.ds(elo * 8, 8), :]
        e0m = jnp.max(jnp.where((row8 >= lo - elo * 8) & (row8 < hi - elo * 8),
                                e0, -jnp.inf), axis=0, keepdims=True)
        e1 = feat_ref[pl.ds(ehi * 8, 8), :]
        e1m = jnp.max(jnp.where((row8 >= lo - ehi * 8) & (row8 < hi - ehi * 8),
                                e1, -jnp.inf), axis=0, keepdims=True)
        mxs.append(jnp.maximum(jnp.maximum(main, e0m), e1m))
    stats_ref[pl.ds(2 * B, B), :] = jnp.maximum(
        stats_ref[pl.ds(2 * B, B), :], jnp.concatenate(mxs, axis=0))


def _pass2_body(seg_ref, feat_ref, stats_ref, w1_ref, w2_ref,
                out_ref, gate_ref):
    i = pl.program_id(0)
    R = feat_ref.shape[0]

    @pl.when(i == 0)
    def _compute_gate():
        st = stats_ref[...]
        K = st.shape[0]
        sums = st[0, 0:B]
        cnts = st[0, B:2 * B, 0:1]
        mx = st[0, 2 * B:3 * B]
        for k in range(1, K):
            sums = sums + st[k, 0:B]
            cnts = cnts + st[k, B:2 * B, 0:1]
            mx = jnp.maximum(mx, st[k, 2 * B:3 * B])
        cnts = jnp.maximum(cnts, 1.0)
        avg = sums / cnts
        mx = jnp.where(jnp.isfinite(mx), mx, 0.0)
        w1 = w1_ref[...]
        w2 = w2_ref[...]

        def mlp(v):
            h = lax.dot_general(v, w1, (((1,), (1,)), ((), ())),
                                preferred_element_type=jnp.float32)
            h = jnp.maximum(h, 0.0)
            return lax.dot_general(h, w2, (((1,), (1,)), ((), ())),
                                   preferred_element_type=jnp.float32)

        z = mlp(avg) + mlp(mx)
        gate_ref[...] = 1.0 / (1.0 + jnp.exp(-z))

    feat = feat_ref[...]
    segv = seg_ref[0, 0, :]
    oh = (lax.broadcasted_iota(jnp.int32, (B, R), 0)
          == segv[None, :]).astype(jnp.float32)
    gr = lax.dot_general(oh, gate_ref[...], (((0,), (0,)), ((), ())),
                         preferred_element_type=jnp.float32)
    out_ref[...] = feat * gr


@jax.jit
def _run(features, W1, W2, segment_ids, batch_size):
    N, C = features.shape
    seg = (segment_ids
           + (jnp.asarray(batch_size) - B).astype(segment_ids.dtype)
           ).astype(jnp.int32)

    R = 6400
    n_tc = N - N_SC
    assert n_tc % R == 0 and N % R == 0
    nb_tc = n_tc // R
    nb = N // R
    seg3 = seg.reshape(nb, 1, R)

    seg_spec = pl.BlockSpec((1, 1, R), lambda i: (i, 0, 0))
    feat_spec = pl.BlockSpec((R, C), lambda i: (i, 0))

    stats_tc = pl.pallas_call(
        _tc_pass1_body,
        grid=(nb_tc,),
        in_specs=[seg_spec, feat_spec],
        out_specs=pl.BlockSpec((3 * B, C), lambda i: (0, 0)),
        out_shape=jax.ShapeDtypeStruct((3 * B, C), jnp.float32),
    )(seg3, features)

    stats_sc = _sc_pass1(features, seg, n_tc, N_SC)

    stats = jnp.concatenate([stats_tc[None], stats_sc], axis=0)

    R2 = 16000
    nb2 = N // R2
    seg3b = seg.reshape(nb2, 1, R2)
    seg_spec2 = pl.BlockSpec((1, 1, R2), lambda i: (i, 0, 0))
    feat_spec2 = pl.BlockSpec((R2, C), lambda i: (i, 0))

    out = pl.pallas_call(
        _pass2_body,
        grid=(nb2,),
        in_specs=[seg_spec2, feat_spec2,
                  pl.BlockSpec((NC + 1, 3 * B, C), lambda i: (0, 0, 0)),
                  pl.BlockSpec((C // 8, C), lambda i: (0, 0)),
                  pl.BlockSpec((C, C // 8), lambda i: (0, 0))],
        out_specs=feat_spec2,
        out_shape=jax.ShapeDtypeStruct((N, C), jnp.float32),
        scratch_shapes=[pltpu.VMEM((B, C), jnp.float32)],
    )(seg3b, features, stats, W1, W2)
    return out


def kernel(features, W1, W2, segment_ids, batch_size):
    return _run(features, W1, W2, segment_ids, batch_size)


# speedup vs baseline: 1.0852x; 1.0852x over previous
import functools

import jax
import jax.numpy as jnp
from jax import lax
from jax.experimental import pallas as pl
from jax.experimental.pallas import tpu as pltpu
from jax.experimental.pallas import tpu_sc as plsc

B = 8
NC = 2
NS = 16
NW = NC * NS
L = 16
N_SC = 128000


def _sc_pass1(features, seg, start, n_sc):
    N, C = features.shape
    G = C // L
    RW = n_sc // NW
    CH = 400
    NCH = RW // CH
    UR = 8
    assert RW % CH == 0 and NCH % 2 == 0 and CH % UR == 0

    mesh = plsc.VectorSubcoreMesh(core_axis_name="c", subcore_axis_name="s",
                                  num_cores=NC, num_subcores=NS)

    @functools.partial(
        pl.kernel,
        out_type=jax.ShapeDtypeStruct((NC, 3 * B, C), jnp.float32),
        mesh=mesh,
        scratch_types=[
            pltpu.VMEM((RW + L,), jnp.int32),
            pltpu.VMEM((CH, C), jnp.float32),
            pltpu.VMEM((CH, C), jnp.float32),
            pltpu.VMEM((3 * B, C), jnp.float32),
            pltpu.VMEM_SHARED((NS, 3 * B, C), jnp.float32),
            pltpu.VMEM((3 * B, C), jnp.float32),
            pltpu.SemaphoreType.DMA,
            pltpu.SemaphoreType.DMA,
        ],
    )
    def pass1(feat_hbm, seg_hbm, stats_o,
              ids_v, buf0, buf1, acc, shr, tmp, sem0, sem1):
        ci = lax.axis_index("c")
        si = lax.axis_index("s")
        wid = si * NC + ci
        base = start + wid * RW

        pltpu.sync_copy(seg_hbm.at[pl.ds(base, RW)], ids_v.at[pl.ds(0, RW)])

        zero = jnp.zeros((L,), jnp.float32)
        ninf = jnp.full((L,), -jnp.inf, jnp.float32)
        for b in range(2 * B):
            for j in range(G):
                acc[b, pl.ds(L * j, L)] = zero
        for b in range(2 * B, 3 * B):
            for j in range(G):
                acc[b, pl.ds(L * j, L)] = ninf

        def process(buf, c):
            s0 = ids_v[pl.ds(c * CH, L)][0]
            s1 = ids_v[pl.ds(c * CH + (CH - L), L)][L - 1]

            @pl.when(s0 == s1)
            def _fast():
                def row_body(rr, carry):
                    fs, fm = carry
                    r0 = rr * UR
                    for k in range(UR):
                        vals = tuple(buf[r0 + k, pl.ds(L * j, L)]
                                     for j in range(G))
                        fs = tuple(fs[j] + vals[j] for j in range(G))
                        fm = tuple(jnp.maximum(fm[j], vals[j])
                                   for j in range(G))
                    return fs, fm

                init = (tuple(zero for _ in range(G)),
                        tuple(ninf for _ in range(G)))
                fs, fm = lax.fori_loop(0, CH // UR, row_body, init)
                for j in range(G):
                    sl = pl.ds(L * j, L)
                    acc[s0, sl] += fs[j]
                    acc[2 * B + s0, sl] = jnp.maximum(acc[2 * B + s0, sl],
                                                      fm[j])
                acc[B + s0, pl.ds(0, L)] += jnp.float32(CH)

            @pl.when(s0 != s1)
            def _boundary():
                def row_body(r, carry):
                    sid = ids_v[pl.ds(c * CH + r, L)][0]
                    for j in range(G):
                        sl = pl.ds(L * j, L)
                        v = buf[r, sl]
                        acc[sid, sl] += v
                        acc[2 * B + sid, sl] = jnp.maximum(
                            acc[2 * B + sid, sl], v)
                    acc[B + sid, pl.ds(0, L)] += 1.0
                    return carry

                lax.fori_loop(0, CH, row_body, 0)

        pltpu.async_copy(feat_hbm.at[pl.ds(base, CH), :], buf0, sem0)

        def chunk_pair(i, carry):
            c0 = i * 2
            c1 = c0 + 1

            pltpu.async_copy(
                feat_hbm.at[pl.ds(base + c1 * CH, CH), :], buf1, sem1)
            pltpu.make_async_copy(
                feat_hbm.at[pl.ds(base, CH), :], buf0, sem0).wait()
            process(buf0, c0)

            @pl.when(c1 + 1 < NCH)
            def _next():
                pltpu.async_copy(
                    feat_hbm.at[pl.ds(base + (c1 + 1) * CH, CH), :],
                    buf0, sem0)

            pltpu.make_async_copy(
                feat_hbm.at[pl.ds(base, CH), :], buf1, sem1).wait()
            process(buf1, c1)
            return carry

        lax.fori_loop(0, NCH // 2, chunk_pair, 0)

        pltpu.sync_copy(acc, shr.at[si])
        plsc.subcore_barrier()

        @pl.when(si == 0)
        def _reduce_and_emit():
            def tile_body(t, carry):
                pltpu.sync_copy(shr.at[t], tmp)
                for b in range(2 * B):
                    for j in range(G):
                        sl = pl.ds(L * j, L)
                        acc[b, sl] += tmp[b, sl]
                for b in range(2 * B, 3 * B):
                    for j in range(G):
                        sl = pl.ds(L * j, L)
                        acc[b, sl] = jnp.maximum(acc[b, sl], tmp[b, sl])
                return carry

            lax.fori_loop(1, NS, tile_body, 0)
            pltpu.sync_copy(acc, stats_o.at[ci])

    return pass1(features, seg)


def _tc_pass1_body(seg_ref, feat_ref, stats_ref):
    i = pl.program_id(0)
    R = feat_ref.shape[0]
    C = feat_ref.shape[1]

    @pl.when(i == 0)
    def _init():
        stats_ref[pl.ds(0, 2 * B), :] = jnp.zeros((2 * B, C), jnp.float32)
        stats_ref[pl.ds(2 * B, B), :] = jnp.full((B, C), -jnp.inf,
                                                 jnp.float32)

    feat = feat_ref[...]
    segv = seg_ref[0, 0, :]

    oh = (lax.broadcasted_iota(jnp.int32, (B, R), 0)
          == segv[None, :]).astype(jnp.float32)
    stats_ref[pl.ds(0, B), :] += lax.dot(oh, feat,
                                         preferred_element_type=jnp.float32)
    stats_ref[pl.ds(B, B), :] += jnp.broadcast_to(
        jnp.sum(oh, axis=1, keepdims=True), (B, C))

    T = R // 8
    m1 = jnp.max(feat.reshape(T, 8, C), axis=1)
    rowt = lax.broadcasted_iota(jnp.int32, (T, C), 0)
    row8 = lax.broadcasted_iota(jnp.int32, (8, C), 0)
    los = [jnp.sum((segv < b).astype(jnp.int32)) for b in range(B)]
    his = los[1:] + [jnp.sum((segv <= B - 1).astype(jnp.int32))]
    mxs = []
    for b in range(B):
        lo, hi = los[b], his[b]
        tlo = (lo + 7) // 8
        thi = hi // 8
        main = jnp.max(jnp.where((rowt >= tlo) & (rowt < thi), m1, -jnp.inf),
                       axis=0, keepdims=True)
        elo = lo // 8
        ehi = jnp.maximum(hi - 1, 0) // 8
        e0 = feat_ref[pl.ds(elo * 8, 8), :]
        e0m = jnp.max(jnp.where((row8 >= lo - elo * 8) & (row8 < hi - elo * 8),
                                e0, -jnp.inf), axis=0, keepdims=True)
        e1 = feat_ref[pl.ds(ehi * 8, 8), :]
        e1m = jnp.max(jnp.where((row8 >= lo - ehi * 8) & (row8 < hi - ehi * 8),
                                e1, -jnp.inf), axis=0, keepdims=True)
        mxs.append(jnp.maximum(jnp.maximum(main, e0m), e1m))
    stats_ref[pl.ds(2 * B, B), :] = jnp.maximum(
        stats_ref[pl.ds(2 * B, B), :], jnp.concatenate(mxs, axis=0))


def _pass2_body(seg_ref, feat_ref, stats_ref, w1_ref, w2_ref,
                out_ref, gate_ref):
    i = pl.program_id(0)
    R = feat_ref.shape[0]

    @pl.when(i == 0)
    def _compute_gate():
        st = stats_ref[...]
        K = st.shape[0]
        sums = st[0, 0:B]
        cnts = st[0, B:2 * B, 0:1]
        mx = st[0, 2 * B:3 * B]
        for k in range(1, K):
            sums = sums + st[k, 0:B]
            cnts = cnts + st[k, B:2 * B, 0:1]
            mx = jnp.maximum(mx, st[k, 2 * B:3 * B])
        cnts = jnp.maximum(cnts, 1.0)
        avg = sums / cnts
        mx = jnp.where(jnp.isfinite(mx), mx, 0.0)
        w1 = w1_ref[...]
        w2 = w2_ref[...]

        def mlp(v):
            h = lax.dot_general(v, w1, (((1,), (1,)), ((), ())),
                                preferred_element_type=jnp.float32)
            h = jnp.maximum(h, 0.0)
            return lax.dot_general(h, w2, (((1,), (1,)), ((), ())),
                                   preferred_element_type=jnp.float32)

        z = mlp(avg) + mlp(mx)
        gate_ref[...] = 1.0 / (1.0 + jnp.exp(-z))

    feat = feat_ref[...]
    segv = seg_ref[0, 0, :]
    oh = (lax.broadcasted_iota(jnp.int32, (B, R), 0)
          == segv[None, :]).astype(jnp.float32)
    gr = lax.dot_general(oh, gate_ref[...], (((0,), (0,)), ((), ())),
                         preferred_element_type=jnp.float32)
    out_ref[...] = feat * gr


@jax.jit
def _run(features, W1, W2, segment_ids, batch_size):
    N, C = features.shape
    seg = (segment_ids
           + (jnp.asarray(batch_size) - B).astype(segment_ids.dtype)
           ).astype(jnp.int32)

    R = 6400
    n_tc = N - N_SC
    assert n_tc % R == 0 and N % R == 0
    nb_tc = n_tc // R
    nb = N // R
    seg3 = seg.reshape(nb, 1, R)

    seg_spec = pl.BlockSpec((1, 1, R), lambda i: (i, 0, 0))
    feat_spec = pl.BlockSpec((R, C), lambda i: (i, 0))

    stats_tc = pl.pallas_call(
        _tc_pass1_body,
        grid=(nb_tc,),
        in_specs=[seg_spec, feat_spec],
        out_specs=pl.BlockSpec((3 * B, C), lambda i: (0, 0)),
        out_shape=jax.ShapeDtypeStruct((3 * B, C), jnp.float32),
    )(seg3, features)

    stats_sc = _sc_pass1(features, seg, n_tc, N_SC)

    stats = jnp.concatenate([stats_tc[None], stats_sc], axis=0)

    R2 = 16000
    nb2 = N // R2
    seg3b = seg.reshape(nb2, 1, R2)
    seg_spec2 = pl.BlockSpec((1, 1, R2), lambda i: (i, 0, 0))
    feat_spec2 = pl.BlockSpec((R2, C), lambda i: (i, 0))

    out = pl.pallas_call(
        _pass2_body,
        grid=(nb2,),
        in_specs=[seg_spec2, feat_spec2,
                  pl.BlockSpec((NC + 1, 3 * B, C), lambda i: (0, 0, 0)),
                  pl.BlockSpec((C // 8, C), lambda i: (0, 0)),
                  pl.BlockSpec((C, C // 8), lambda i: (0, 0))],
        out_specs=feat_spec2,
        out_shape=jax.ShapeDtypeStruct((N, C), jnp.float32),
        scratch_shapes=[pltpu.VMEM((B, C), jnp.float32)],
    )(seg3b, features, stats, W1, W2)
    return out


def kernel(features, W1, W2, segment_ids, batch_size):
    return _run(features, W1, W2, segment_ids, batch_size)
